# baseline (speedup 1.0000x reference)
import functools

import jax
import jax.numpy as jnp
from jax import lax
from jax.experimental import pallas as pl
from jax.experimental.pallas import tpu as pltpu
from jax.experimental.pallas import tpu_sc as plsc

N_NODES = 100000
DIM = 128
NUM_GRAPHS = 512

NC = 2
NS = 16
NW = NC * NS

BLK = 128
NFULL = N_NODES // BLK
TAIL = N_NODES - NFULL * BLK
ITERS = (NFULL + NW - 1) // NW
NBLK_PAD = NW * ITERS
GROWS = NUM_GRAPHS // NS

_mesh = plsc.VectorSubcoreMesh(core_axis_name="c", subcore_axis_name="s")


@functools.partial(
    pl.kernel,
    out_type=(
        jax.ShapeDtypeStruct((NC, NUM_GRAPHS, DIM), jnp.float32),
        jax.ShapeDtypeStruct((NW, NUM_GRAPHS), jnp.float32),
    ),
    mesh=_mesh,
    compiler_params=pltpu.CompilerParams(needs_layout_passes=False),
    scratch_types=[
        pltpu.VMEM((BLK, DIM), jnp.float32),
        pltpu.VMEM((BLK, DIM), jnp.float32),
        pltpu.VMEM((ITERS, 1, BLK), jnp.int32),
        pltpu.VMEM((TAIL, DIM), jnp.float32),
        pltpu.VMEM((TAIL,), jnp.int32),
        pltpu.VMEM((NUM_GRAPHS,), jnp.float32),
        pltpu.VMEM_SHARED((NUM_GRAPHS, DIM), jnp.float32),
        pltpu.SemaphoreType.DMA,
        pltpu.SemaphoreType.DMA,
        pltpu.SemaphoreType.DMA,
        pltpu.SemaphoreType.DMA,
        pltpu.SemaphoreType.DMA,
    ],
)
def _segment_pool(feat_hbm, ids_hbm, ids3_hbm, zsum_hbm, zl_hbm,
                  psums_hbm, lout_hbm,
                  feat_a, feat_b, ids_all_v, feat_t, ids_t, ltab_v,
                  sums_sp, fsem0, fsem1, ssem0, ssem1, isem):
    cid = lax.axis_index("c")
    sid = lax.axis_index("s")
    wid = cid * NS + sid
    fsems = (fsem0, fsem1)
    ssems = (ssem0, ssem1)
    feats = (feat_a, feat_b)

    iota16 = lax.iota(jnp.int32, 16)
    blk0 = wid * ITERS

    ids_dma = pltpu.async_copy(ids3_hbm.at[pl.ds(blk0, ITERS)], ids_all_v,
                               isem)
    pltpu.sync_copy(zl_hbm, ltab_v)
    pltpu.sync_copy(zsum_hbm.at[pl.ds(sid * GROWS, GROWS)],
                    sums_sp.at[pl.ds(sid * GROWS, GROWS)])
    ids_dma.wait()
    plsc.subcore_barrier()

    def _positions(base, j):
        return (base + (j * 16 + 1) + iota16).astype(jnp.float32)

    def _start_load(i):
        blk = blk0 + i
        slot = i & 1
        base = pl.multiple_of(blk * BLK, BLK)
        desc = [None]

        @pl.when(blk < NFULL)
        def _():
            desc[0] = pltpu.async_copy(
                feat_hbm.at[pl.ds(base, BLK)], feats[slot], fsems[slot])
        return desc

    pending = _start_load(0)
    scat = [None, None]
    for i in range(ITERS):
        blk = blk0 + i
        slot = i & 1
        nslot = slot ^ 1
        if scat[nslot] is not None:
            prev_blk = blk0 + i - 1

            @pl.when(prev_blk < NFULL)
            def _():
                scat[nslot][0].wait()
            scat[nslot] = None
        nxt = _start_load(i + 1) if i + 1 < ITERS else None
        cur = pending

        @pl.when(blk < NFULL)
        def _():
            cur[0].wait()
            base = blk * BLK
            scat[slot] = [pltpu.async_copy(
                feats[slot], sums_sp.at[ids_all_v.at[i, 0]], ssems[slot],
                add=True)]
            for j in range(BLK // 16):
                idx = ids_all_v[i, 0, pl.ds(j * 16, 16)]
                plsc.store_scatter(ltab_v, [idx], _positions(base, j))
        pending = nxt

    for s in range(2):
        if scat[s] is not None:
            last_blk = blk0 + ITERS - 1

            @pl.when(last_blk < NFULL)
            def _():
                scat[s][0].wait()

    @pl.when(wid == NW - 1)
    def _():
        base = NFULL * BLK
        pltpu.sync_copy(feat_hbm.at[pl.ds(base, TAIL)], feat_t)
        pltpu.sync_copy(ids_hbm.at[pl.ds(base, TAIL)], ids_t)
        pltpu.sync_copy(feat_t, sums_sp.at[ids_t], add=True)
        for j in range(TAIL // 16):
            idx = ids_t[pl.ds(j * 16, 16)]
            plsc.store_scatter(ltab_v, [idx], _positions(base, j))

    plsc.subcore_barrier()

    pltpu.sync_copy(sums_sp.at[pl.ds(sid * GROWS, GROWS)],
                    psums_hbm.at[cid, pl.ds(sid * GROWS, GROWS)])
    pltpu.sync_copy(ltab_v, lout_hbm.at[wid])


def _head_body(ps_ref, lt_ref, w_ref, b_ref, out_ref):
    lpos = jnp.max(lt_ref[...], axis=0)
    gi = lax.broadcasted_iota(jnp.int32, (NUM_GRAPHS, NUM_GRAPHS), 0)
    gj = lax.broadcasted_iota(jnp.int32, (NUM_GRAPHS, NUM_GRAPHS), 1)
    prev = jnp.max(jnp.where(gi < gj, lpos[:, None], 0.0), axis=0)
    cnt = jnp.maximum(lpos - prev, 1.0)[:, None]
    emb = (ps_ref[0] + ps_ref[1]) / cnt
    out_ref[...] = (
        jnp.dot(emb, w_ref[...], preferred_element_type=jnp.float32)
        + b_ref[...]
    )


def _head(psums, lout, W, b2d):
    return pl.pallas_call(
        _head_body,
        out_shape=jax.ShapeDtypeStruct((NUM_GRAPHS, DIM), jnp.float32),
    )(psums, lout, W, b2d)


@jax.jit
def kernel(node_feature, batch_ids, graph_label, W, b):
    ids = batch_ids.astype(jnp.int32)
    ids3 = jnp.pad(ids, (0, NBLK_PAD * BLK - N_NODES)).reshape(
        NBLK_PAD, 1, BLK)
    zsum = jnp.zeros((NUM_GRAPHS, DIM), jnp.float32)
    zl = jnp.zeros((NUM_GRAPHS,), jnp.float32)
    psums, lout = _segment_pool(node_feature, ids, ids3, zsum, zl)
    pred = _head(psums, lout, W, b.reshape(1, DIM))
    return (pred, graph_label)


# speedup vs baseline: 1.0806x; 1.0806x over previous
import functools

import jax
import jax.numpy as jnp
from jax import lax
from jax.experimental import pallas as pl
from jax.experimental.pallas import tpu as pltpu
from jax.experimental.pallas import tpu_sc as plsc

N_NODES = 100000
DIM = 128
NUM_GRAPHS = 512

NC = 2
NS = 16
NW = NC * NS

BLK = 128
NFULL = N_NODES // BLK
TAIL = N_NODES - NFULL * BLK
ITERS = (NFULL + NW - 1) // NW
NBLK_PAD = NW * ITERS
GROWS = NUM_GRAPHS // NS

_mesh = plsc.VectorSubcoreMesh(core_axis_name="c", subcore_axis_name="s")


@functools.partial(
    pl.kernel,
    out_type=(
        jax.ShapeDtypeStruct((NC, NUM_GRAPHS, DIM), jnp.float32),
        jax.ShapeDtypeStruct((NW, NUM_GRAPHS), jnp.float32),
    ),
    mesh=_mesh,
    compiler_params=pltpu.CompilerParams(needs_layout_passes=False),
    scratch_types=[
        pltpu.VMEM((BLK, DIM), jnp.float32),
        pltpu.VMEM((BLK, DIM), jnp.float32),
        pltpu.VMEM((BLK,), jnp.int32),
        pltpu.VMEM((BLK,), jnp.int32),
        pltpu.VMEM((TAIL, DIM), jnp.float32),
        pltpu.VMEM((TAIL,), jnp.int32),
        pltpu.VMEM((NUM_GRAPHS,), jnp.float32),
        pltpu.VMEM_SHARED((NUM_GRAPHS, DIM), jnp.float32),
        pltpu.SemaphoreType.DMA,
        pltpu.SemaphoreType.DMA,
        pltpu.SemaphoreType.DMA,
        pltpu.SemaphoreType.DMA,
        pltpu.SemaphoreType.DMA,
        pltpu.SemaphoreType.DMA,
    ],
)
def _segment_pool(feat_hbm, ids_hbm, zsum_hbm, zl_hbm,
                  psums_hbm, lout_hbm,
                  feat_a, feat_b, ids_a, ids_b, feat_t, ids_t, ltab_v,
                  sums_sp, fsem0, fsem1, ssem0, ssem1, isem0, isem1):
    cid = lax.axis_index("c")
    sid = lax.axis_index("s")
    wid = cid * NS + sid
    fsems = (fsem0, fsem1)
    ssems = (ssem0, ssem1)
    isems = (isem0, isem1)
    feats = (feat_a, feat_b)
    idss = (ids_a, ids_b)

    iota16 = lax.iota(jnp.int32, 16)

    pltpu.sync_copy(zl_hbm, ltab_v)
    pltpu.sync_copy(zsum_hbm.at[pl.ds(sid * GROWS, GROWS)],
                    sums_sp.at[pl.ds(sid * GROWS, GROWS)])
    plsc.subcore_barrier()

    def _positions(base, j):
        return (base + (j * 16 + 1) + iota16).astype(jnp.float32)

    def _start_load(i):
        blk = i * NW + wid
        slot = i & 1
        base = pl.multiple_of(blk * BLK, BLK)
        desc = [None, None]

        @pl.when(blk < NFULL)
        def _():
            desc[0] = pltpu.async_copy(
                feat_hbm.at[pl.ds(base, BLK)], feats[slot], fsems[slot])
            desc[1] = pltpu.async_copy(
                ids_hbm.at[pl.ds(base, BLK)], idss[slot], isems[slot])
        return desc

    pending = _start_load(0)
    scat = [None, None]
    for i in range(ITERS):
        blk = i * NW + wid
        slot = i & 1
        nslot = slot ^ 1
        if scat[nslot] is not None:
            prev_blk = (i - 1) * NW + wid

            @pl.when(prev_blk < NFULL)
            def _():
                scat[nslot][0].wait()
            scat[nslot] = None
        nxt = _start_load(i + 1) if i + 1 < ITERS else None
        cur = pending

        @pl.when(blk < NFULL)
        def _():
            cur[0].wait()
            cur[1].wait()
            base = blk * BLK
            scat[slot] = [pltpu.async_copy(
                feats[slot], sums_sp.at[idss[slot]], ssems[slot],
                add=True)]
            for j in range(BLK // 16):
                idx = idss[slot][pl.ds(j * 16, 16)]
                plsc.store_scatter(ltab_v, [idx], _positions(base, j))
        pending = nxt

    for sdrain in range(2):
        if scat[sdrain] is not None:
            last_blk = (ITERS - 1) * NW + wid

            @pl.when(last_blk < NFULL)
            def _():
                scat[sdrain][0].wait()

    @pl.when(wid == NW - 1)
    def _():
        base = NFULL * BLK
        pltpu.sync_copy(feat_hbm.at[pl.ds(base, TAIL)], feat_t)
        pltpu.sync_copy(ids_hbm.at[pl.ds(base, TAIL)], ids_t)
        pltpu.sync_copy(feat_t, sums_sp.at[ids_t], add=True)
        for j in range(TAIL // 16):
            idx = ids_t[pl.ds(j * 16, 16)]
            plsc.store_scatter(ltab_v, [idx], _positions(base, j))

    plsc.subcore_barrier()

    pltpu.sync_copy(sums_sp.at[pl.ds(sid * GROWS, GROWS)],
                    psums_hbm.at[cid, pl.ds(sid * GROWS, GROWS)])
    pltpu.sync_copy(ltab_v, lout_hbm.at[wid])


def _head_body(ps_ref, lt_ref, w_ref, b_ref, out_ref):
    lpos = jnp.max(lt_ref[...], axis=0)
    gi = lax.broadcasted_iota(jnp.int32, (NUM_GRAPHS, NUM_GRAPHS), 0)
    gj = lax.broadcasted_iota(jnp.int32, (NUM_GRAPHS, NUM_GRAPHS), 1)
    prev = jnp.max(jnp.where(gi < gj, lpos[:, None], 0.0), axis=0)
    cnt = jnp.maximum(lpos - prev, 1.0)[:, None]
    emb = (ps_ref[0] + ps_ref[1]) / cnt
    out_ref[...] = (
        jnp.dot(emb, w_ref[...], preferred_element_type=jnp.float32)
        + b_ref[...]
    )


def _head(psums, lout, W, b2d):
    return pl.pallas_call(
        _head_body,
        out_shape=jax.ShapeDtypeStruct((NUM_GRAPHS, DIM), jnp.float32),
    )(psums, lout, W, b2d)


@jax.jit
def kernel(node_feature, batch_ids, graph_label, W, b):
    ids = batch_ids.astype(jnp.int32)
    zsum = jnp.zeros((NUM_GRAPHS, DIM), jnp.float32)
    zl = jnp.zeros((NUM_GRAPHS,), jnp.float32)
    psums, lout = _segment_pool(node_feature, ids, zsum, zl)
    pred = _head(psums, lout, W, b.reshape(1, DIM))
    return (pred, graph_label)
